# baseline (speedup 1.0000x reference)
import jax
import jax.numpy as jnp
from jax import lax
from jax.experimental import pallas as pl
from jax.experimental.pallas import tpu as pltpu

R, SUB, LANE = 128, 8, 4096
G = 4
NG = R // G
W = 3
NBUF = 2


def _body(inds_ref, x_ref, out_ref, buf_ref, aux_ref, viol_ref, sem_ref, aux_sem_ref):
    viol_ref[0] = jnp.float32(0.0)

    def group_scalars(g):
        base = g * G
        m = [inds_ref[base + k] <= 0 for k in range(G)]
        any_m = m[0] | m[1] | m[2] | m[3]
        lo = jnp.where(m[0], 0, jnp.where(m[1], 1, jnp.where(m[2], 2, 3)))
        hi = jnp.where(m[3], 3, jnp.where(m[2], 2, jnp.where(m[1], 1, 0)))
        start = jnp.minimum(base + lo, R - W)
        hi_row = base + hi
        need_aux = any_m & (hi_row > start + (W - 1))
        return any_m, start, hi_row, need_aux

    def issue(g, slot):
        any_m, start, hi_row, need_aux = group_scalars(g)

        @pl.when(any_m)
        def _():
            pltpu.make_async_copy(
                x_ref.at[pl.ds(start, W)], buf_ref.at[slot], sem_ref.at[slot]
            ).start()

        @pl.when(need_aux)
        def _():
            pltpu.make_async_copy(
                x_ref.at[hi_row], aux_ref.at[slot], aux_sem_ref.at[slot]
            ).start()

    issue(0, 0)

    def body(g, carry):
        slot = lax.rem(g, NBUF)

        @pl.when(g + 1 < NG)
        def _():
            issue(g + 1, lax.rem(g + 1, NBUF))

        any_m, start, hi_row, need_aux = group_scalars(g)

        @pl.when(any_m)
        def _():
            pltpu.make_async_copy(
                x_ref.at[pl.ds(start, W)], buf_ref.at[slot], sem_ref.at[slot]
            ).wait()
            cnt = jnp.float32(0.0)
            for k in range(W):
                mk = jnp.where(inds_ref[start + k] <= 0, 1.0, 0.0).astype(jnp.float32)
                v = buf_ref[slot, k]
                cnt = cnt + mk * jnp.sum(jnp.where(v != v, 1.0, 0.0).astype(jnp.float32))
            viol_ref[0] = viol_ref[0] + cnt

        @pl.when(need_aux)
        def _():
            pltpu.make_async_copy(
                x_ref.at[hi_row], aux_ref.at[slot], aux_sem_ref.at[slot]
            ).wait()
            va = aux_ref[slot]
            viol_ref[0] = viol_ref[0] + jnp.sum(
                jnp.where(va != va, 1.0, 0.0).astype(jnp.float32)
            )

        return carry

    lax.fori_loop(0, NG, body, 0)

    out_ref[...] = jnp.where(viol_ref[0] == 0.0, 1.0, 0.0).astype(jnp.float32) * jnp.ones(
        (1, 1), jnp.float32
    )


def kernel(x, inds):
    r, c = x.shape
    inds2 = jnp.asarray(inds, dtype=jnp.int32)
    x3 = x.reshape(r, SUB, c // SUB)
    out = pl.pallas_call(
        _body,
        in_specs=[
            pl.BlockSpec(memory_space=pltpu.SMEM),
            pl.BlockSpec(memory_space=pltpu.MemorySpace.HBM),
        ],
        out_specs=pl.BlockSpec(memory_space=pltpu.VMEM),
        out_shape=jax.ShapeDtypeStruct((1, 1), jnp.float32),
        scratch_shapes=[
            pltpu.VMEM((NBUF, W, SUB, LANE), jnp.float32),
            pltpu.VMEM((NBUF, SUB, LANE), jnp.float32),
            pltpu.SMEM((1,), jnp.float32),
            pltpu.SemaphoreType.DMA((NBUF,)),
            pltpu.SemaphoreType.DMA((NBUF,)),
        ],
    )(inds2, x3)
    return out.reshape(1)
